# Initial kernel scaffold; baseline (speedup 1.0000x reference)
import jax
import jax.numpy as jnp
from jax.experimental import pallas as pl


def kernel(x_enc, x_mark_enc, x_dec, x_mark_dec, params):
    raise NotImplementedError("write your pallas kernel here")


# speedup vs baseline: 5.1209x; 5.1209x over previous
import numpy as np
import jax
import jax.numpy as jnp
from jax.experimental import pallas as pl
from jax.experimental.pallas import tpu as pltpu

_B, _L, _C, _D, _PRED, _NH, _DFF, _NL, _BOT = 4, 512, 7, 128, 96, 8, 256, 2, 16
_DH = _D // _NH


def _posemb_np():
    pos = np.arange(_L)[:, None].astype(np.float32)
    div = np.exp(np.arange(0, _D, 2).astype(np.float32) * (-np.log(10000.0) / _D))
    pe = np.zeros((_L, _D), np.float32)
    pe[:, 0::2] = np.sin(pos * div)
    pe[:, 1::2] = np.cos(pos * div)
    return pe


_PE = _posemb_np()


def _ln(x, g, b):
    m = jnp.mean(x, axis=1, keepdims=True)
    v = jnp.mean((x - m) ** 2, axis=1, keepdims=True)
    return (x - m) / jnp.sqrt(v + 1e-5) * g + b


def _row_kernel(*refs):
    (xr_ref, ve1w_ref, ve1b_ref, ve2w_ref, ve2b_ref, rpw_ref, rpb_ref,
     pe_ref) = refs[:8]
    layer_refs = refs[8:8 + 16 * _NL]
    (bng_ref, bnb_ref, bnrm_ref, bnrv_ref, oh1w_ref, oh1b_ref) = refs[8 + 16 * _NL:8 + 16 * _NL + 6]
    (t_ref, pb_ref, mean_ref, std_ref) = refs[8 + 16 * _NL + 6:]

    f32 = jnp.float32
    x = xr_ref[0]

    m = jnp.mean(x, axis=1, keepdims=True)
    xc = x - m
    m2 = jnp.mean(xc, axis=1, keepdims=True)
    var = jnp.mean((xc - m2) ** 2, axis=1, keepdims=True)
    std = jnp.sqrt(var + 1e-5)
    xn = xc / std

    bf16 = jnp.bfloat16
    v = jnp.dot(xn.astype(bf16), ve1w_ref[...].astype(bf16),
                preferred_element_type=f32) + ve1b_ref[...]
    v16 = v.astype(bf16).astype(f32)
    h = jnp.zeros((_L, _D), f32)
    for k in range(16):
        h = h + ve2w_ref[k].astype(bf16).astype(f32) * v16[0:1, k:k + 1]
    h = h + ve2b_ref[...]

    resid = jnp.dot(h, rpw_ref[...]) + rpb_ref[...]

    hq = h.astype(bf16).astype(f32)
    nrm = jnp.sqrt(jnp.sum(hq * hq, axis=1, keepdims=True))
    dots = jnp.sum(hq[:_L - 1] * hq[1:], axis=1, keepdims=True)
    cos = dots / (nrm[:_L - 1] * nrm[1:] + 1e-12)
    del hq
    pbt = jnp.clip((1.0 - cos) * 0.5, 0.0, 1.0)
    pb = jnp.concatenate([jnp.ones((1, 1), f32), pbt], axis=0)

    isub = jax.lax.broadcasted_iota(jnp.int32, (_L, 1), 0)
    bmaskf = jnp.where(pb >= 0.5, 1.0, 0.0)
    bmaskf = jnp.where(isub == 0, 1.0, bmaskf)

    ii = jax.lax.broadcasted_iota(jnp.int32, (_L, _L), 0).astype(f32)
    jj = jax.lax.broadcasted_iota(jnp.int32, (_L, _L), 1).astype(f32)
    tri = jnp.where(jj <= ii, 1.0, 0.0)
    c_s = jnp.dot(tri, bmaskf)

    eye = jnp.where(ii == jj, 1.0, 0.0)
    ones_row = jnp.ones((1, _L), f32)
    b_l = jnp.dot(ones_row, eye * bmaskf)
    c_l = jnp.dot(b_l, jnp.where(ii <= jj, 1.0, 0.0))

    P = jnp.where((jnp.abs(c_l - 1.0 - ii) < 0.5) & (b_l > 0.5), 1.0, 0.0)
    G = jnp.where(jnp.abs(c_s - 1.0 - jj) < 0.5, 1.0, 0.0)

    z = jnp.dot(P, h) + pe_ref[...]

    inv_sqrt_dh = 1.0 / np.sqrt(float(_DH))
    for l in range(_NL):
        (wq, bq, wk, bk, wv, bv, wo, bo, c1w, c1b, c2w, c2b,
         g1, b1, g2, b2) = (r[...] for r in layer_refs[16 * l:16 * (l + 1)])
        q = jnp.dot(z, wq) + bq
        kk = jnp.dot(z, wk) + bk
        vv = jnp.dot(z, wv) + bv
        outs = []
        for hh in range(_NH):
            sl = slice(hh * _DH, (hh + 1) * _DH)
            s = jax.lax.dot_general(q[:, sl], kk[:, sl],
                                    (((1,), (1,)), ((), ()))) * inv_sqrt_dh
            s = s - jnp.max(s, axis=1, keepdims=True)
            e = jnp.exp(s)
            a = e / jnp.sum(e, axis=1, keepdims=True)
            outs.append(jnp.dot(a, vv[:, sl]))
        o = jnp.concatenate(outs, axis=1)
        o = jnp.dot(o, wo) + bo
        x1 = _ln(z + o, g1, b1)
        y = jax.nn.gelu(jnp.dot(x1, c1w) + c1b)
        y = jnp.dot(y, c2w) + c2b
        z = _ln(x1 + y, g2, b2)

    z = (z - bnrm_ref[...]) / jnp.sqrt(bnrv_ref[...] + 1e-5) * bng_ref[...] + bnb_ref[...]

    expanded = jnp.dot(G, z)

    w = jnp.clip(pb, 1e-4, 1.0)
    bb = w * expanded
    aa = 1.0 - w
    d = 1
    while d < _L:
        a_sh = jnp.concatenate([jnp.ones((d, 1), f32), aa[:_L - d]], axis=0)
        b_sh = jnp.concatenate([jnp.zeros((d, _D), f32), bb[:_L - d]], axis=0)
        bb = aa * b_sh + bb
        aa = aa * a_sh
        d *= 2
    hs = bb + resid

    t = jnp.dot(jnp.transpose(hs), oh1w_ref[...]) + oh1b_ref[...]

    t_ref[...] = t.reshape(1, _D, _BOT)
    pb_ref[...] = pb.reshape(1, _L, 1)
    mean_ref[...] = jnp.broadcast_to(m.reshape(1, 1, 1), (1, 1, 128))
    std_ref[...] = jnp.broadcast_to(std.reshape(1, 1, 1), (1, 1, 128))


def _head_kernel(t_ref, w_ref, b_ref, mu_ref, sd_ref, out_ref):
    t = t_ref[...]
    dec = jnp.dot(t, w_ref[...]) + b_ref[...]
    out_ref[...] = dec * sd_ref[...] + mu_ref[...]


def kernel(x_enc, x_mark_enc, x_dec, x_mark_dec, params):
    p = params
    bc = _B * _C
    f32 = jnp.float32

    xr = jnp.transpose(x_enc, (0, 2, 1)).reshape(bc, 1, _L).astype(f32)

    def rv(a, n):
        return a.reshape(1, n)

    operands = [
        xr,
        p['ve1_w'], rv(p['ve1_b'], _BOT),
        p['ve2_w'].reshape(_BOT, _L, _D), p['ve2_b'].reshape(_L, _D),
        p['rp_w'], rv(p['rp_b'], _D),
        jnp.asarray(_PE),
    ]
    for lp in p['layers']:
        operands += [
            lp['wq'], rv(lp['bq'], _D), lp['wk'], rv(lp['bk'], _D),
            lp['wv'], rv(lp['bv'], _D), lp['wo'], rv(lp['bo'], _D),
            lp['c1_w'], rv(lp['c1_b'], _DFF), lp['c2_w'], rv(lp['c2_b'], _D),
            rv(lp['ln1_g'], _D), rv(lp['ln1_b'], _D),
            rv(lp['ln2_g'], _D), rv(lp['ln2_b'], _D),
        ]
    operands += [
        rv(p['bn_g'], _D), rv(p['bn_b'], _D),
        rv(p['bn_rm'], _D), rv(p['bn_rv'], _D),
        p['oh1_w'], rv(p['oh1_b'], _BOT),
    ]

    def full_spec(a):
        nd = a.ndim
        return pl.BlockSpec(a.shape, lambda i, _n=nd: (0,) * _n)

    in_specs = [pl.BlockSpec((1, 1, _L), lambda i: (i, 0, 0))]
    in_specs += [full_spec(a) for a in operands[1:]]

    out_shapes = [
        jax.ShapeDtypeStruct((bc, _D, _BOT), f32),
        jax.ShapeDtypeStruct((bc, _L, 1), f32),
        jax.ShapeDtypeStruct((bc, 1, 128), f32),
        jax.ShapeDtypeStruct((bc, 1, 128), f32),
    ]
    out_specs = [
        pl.BlockSpec((1, _D, _BOT), lambda i: (i, 0, 0)),
        pl.BlockSpec((1, _L, 1), lambda i: (i, 0, 0)),
        pl.BlockSpec((1, 1, 128), lambda i: (i, 0, 0)),
        pl.BlockSpec((1, 1, 128), lambda i: (i, 0, 0)),
    ]

    t_all, pb_all, mean_all, std_all = pl.pallas_call(
        _row_kernel,
        grid=(bc,),
        in_specs=in_specs,
        out_specs=out_specs,
        out_shape=out_shapes,
        compiler_params=pltpu.CompilerParams(
            dimension_semantics=("arbitrary",)),
    )(*operands)

    t_flat = t_all.reshape(bc, _D * _BOT)
    mu = mean_all[:, 0, 0:1]
    sd = std_all[:, 0, 0:1]

    dec_all = pl.pallas_call(
        _head_kernel,
        out_shape=jax.ShapeDtypeStruct((bc, _PRED), f32),
    )(t_flat, p['oh2_w'], rv(p['oh2_b'], _PRED), mu, sd)

    dec_out = jnp.transpose(dec_all.reshape(_B, _C, _PRED), (0, 2, 1))

    pb_bc = pb_all.reshape(_B, _C, _L)
    bmask = pb_bc >= 0.5
    bmask = bmask.at[:, :, 0].set(True)
    boundary_prob = jnp.stack([1.0 - pb_bc, pb_bc], axis=-1)
    selected = jnp.where(bmask, pb_bc, 1.0 - pb_bc)[..., None]

    return dec_out, bmask, boundary_prob, selected
